# baseline (speedup 1.0000x reference)
import functools

import jax
import jax.numpy as jnp
from jax import lax
from jax.experimental import pallas as pl
from jax.experimental.pallas import tpu as pltpu
from jax.experimental.pallas import tpu_sc as plsc

KS = 5
DIM = 3
K = 125
KPAD = 128
N = 2048
E = 8192
J = E * 8
F = 64
IN0P = 16
LAYERS = 6

RT_TILES = 16
DATA_TILES = 640
T_TOT = RT_TILES + DATA_TILES
C = T_TOT * 128
NPAD = 16384
EBLK = 1024
JBLK = 2048
NJB = J // JBLK

_NC = 2
_NS = 16
_NW = _NC * _NS
_JPW = C // _NW
_CH = _JPW // 4
_NPT = N // _NS

_VM = pltpu.CompilerParams(vmem_limit_bytes=100 * 1024 * 1024)


def _prep1_body(ei_ref, attr_ref, bas_ref, widx_ref, srep_ref, drep_ref,
                deg_ref):
    i = pl.program_id(0)
    eiT = jnp.transpose(ei_ref[...])
    srcc = eiT[:, 0:1]
    dstc = eiT[:, 1:2]
    dst_row = ei_ref[1:2, :]

    p = attr_ref[...] * (KS - 1.0)
    lo = jnp.floor(p)
    frac = p - lo
    lo_i = jnp.clip(lo.astype(jnp.int32), 0, KS - 1)

    bidx = lax.broadcasted_iota(jnp.int32, (8, 1), 0)
    basis8 = jnp.ones((8, EBLK), jnp.float32)
    widx8 = jnp.zeros((8, EBLK), jnp.int32)
    for d in range(DIM):
        bi = (bidx >> d) & 1
        bf = bi.astype(jnp.float32)
        f = frac[d:d + 1, :]
        basis8 = basis8 * (bf * f + (1.0 - bf) * (1.0 - f))
        ii = jnp.clip(lo_i[d:d + 1, :] + bi, 0, KS - 1)
        widx8 = widx8 * KS + ii

    bas_ref[...] = jnp.transpose(basis8)
    widx_ref[...] = jnp.transpose(widx8)
    srep_ref[...] = jnp.broadcast_to(srcc, (EBLK, 8))
    drep_ref[...] = jnp.broadcast_to(dstc, (EBLK, 8))

    nio0 = lax.broadcasted_iota(jnp.int32, (N, EBLK), 0)
    part = jnp.sum((dst_row == nio0).astype(jnp.float32), axis=1,
                   keepdims=True)

    @pl.when(i == 0)
    def _():
        deg_ref[...] = jnp.zeros_like(deg_ref)

    deg_ref[...] += part


def _prep1b_body(ei_ref, deg_ref, bas_ref, out_ref):
    eiT = jnp.transpose(ei_ref[...])
    dstc = eiT[:, 1:2]
    nio = lax.broadcasted_iota(jnp.int32, (EBLK, N), 1)
    Dblk = (dstc == nio).astype(jnp.float32)
    invd = 1.0 / jnp.maximum(deg_ref[...], 1.0)
    hi = invd.astype(jnp.bfloat16).astype(jnp.float32)
    lo = invd - hi
    inv_e = (jnp.dot(Dblk, hi, preferred_element_type=jnp.float32)
             + jnp.dot(Dblk, lo, preferred_element_type=jnp.float32))
    out_ref[...] = bas_ref[...] * inv_e


def _prep2_body(key_ref, ts_ref, rank_ref, cnt_ref):
    kio = lax.broadcasted_iota(jnp.int32, (JBLK, KPAD), 1)
    oh = (key_ref[...] == kio).astype(jnp.float32)
    tsoh = jnp.dot(ts_ref[...], oh, preferred_element_type=jnp.float32)
    rank_ref[...] = jnp.sum(tsoh * oh, axis=1, keepdims=True)
    cnt_ref[...] = jnp.sum(oh, axis=0, keepdims=True)[None]


def _prep4_body(cnt_ref, ts128_ref, soff_ref, tk_ref, ppad_ref):
    lane = lax.broadcasted_iota(jnp.int32, (1, KPAD), 1)
    total = jnp.sum(cnt_ref[...], axis=0)
    t_i = total.astype(jnp.int32)
    ntc = (t_i + 127) >> 7
    s123 = jnp.sum(jnp.where(lane <= 123, ntc, 0))
    nt = jnp.where(lane == 124, DATA_TILES - s123,
                   jnp.where(lane <= 123, ntc, 0))
    ntf = nt.astype(jnp.float32)

    def exact_prefix(v_i32):
        d0 = (v_i32 & 255).astype(jnp.float32)
        d1 = ((v_i32 >> 8) & 255).astype(jnp.float32)
        d2 = (v_i32 >> 16).astype(jnp.float32)
        p = jnp.dot(d0, ts128_ref[...], preferred_element_type=jnp.float32)
        p += 256.0 * jnp.dot(d1, ts128_ref[...],
                             preferred_element_type=jnp.float32)
        p += 65536.0 * jnp.dot(d2, ts128_ref[...],
                               preferred_element_type=jnp.float32)
        return p

    tsf = RT_TILES + exact_prefix(nt)
    soff = 128.0 * tsf
    soff_ref[...] = soff

    tio = lax.broadcasted_iota(jnp.int32, (T_TOT + 112, 1), 0)
    ends = tsf + ntf
    cntk = jnp.sum((ends <= tio.astype(jnp.float32)).astype(jnp.float32),
                   axis=1, keepdims=True)
    tk = jnp.where(tio < RT_TILES, K,
                   jnp.minimum(cntk.astype(jnp.int32), KPAD - 1))
    tk_ref[...] = tk

    padk = 128.0 * ntf - total * (lane <= 124)
    padst = exact_prefix(padk.astype(jnp.int32))
    padend = padst + padk
    cio = lax.broadcasted_iota(jnp.int32, (NPAD, 1), 0).astype(jnp.float32)
    kc = jnp.sum((padend <= cio).astype(jnp.float32), axis=1,
                 keepdims=True)
    kio2 = lax.broadcasted_iota(jnp.int32, (NPAD, KPAD), 1)
    ohc = (kc.astype(jnp.int32) == kio2).astype(jnp.float32)
    ps_c = jnp.sum(ohc * padst, axis=1, keepdims=True)
    cnt_c = jnp.sum(ohc * total, axis=1, keepdims=True)
    so_c = jnp.sum(ohc * soff, axis=1, keepdims=True)
    ppad_ref[...] = (so_c + cnt_c + (cio - ps_c)).astype(jnp.int32)


def _prep3_body(key_ref, rank_ref, cnt_ref, soff_ref, pos_ref):
    i = pl.program_id(0)
    kio = lax.broadcasted_iota(jnp.int32, (JBLK, KPAD), 1)
    oh = (key_ref[...] == kio).astype(jnp.float32)
    rmask = (lax.broadcasted_iota(jnp.int32, (NJB, 1, 1), 0)
             < i).astype(jnp.float32)
    bp = jnp.sum(cnt_ref[...] * rmask, axis=0)
    base = soff_ref[...] + bp
    pos = jnp.sum(oh * base, axis=1, keepdims=True) + rank_ref[...]
    pos_ref[...] = pos.astype(jnp.int32)


@functools.cache
def _smesh():
    return plsc.VectorSubcoreMesh(core_axis_name="c", subcore_axis_name="s")


@functools.cache
def _make_sc_sortscat():
    return functools.partial(
        pl.kernel, mesh=_smesh(),
        out_type=[
            jax.ShapeDtypeStruct((C,), jnp.int32),
            jax.ShapeDtypeStruct((C,), jnp.int32),
            jax.ShapeDtypeStruct((C,), jnp.float32),
        ],
        scratch_types=[
            pltpu.VMEM((_JPW,), jnp.int32),
            pltpu.VMEM((_JPW,), jnp.int32),
            pltpu.VMEM((_JPW,), jnp.int32),
            pltpu.VMEM((_JPW,), jnp.float32),
            pltpu.SemaphoreType.DMA,
        ],
    )(_sc_sortscat_body)


def _sc_sortscat_body(pos_h, src_h, dst_h, bas_h, src_o, dst_o, bas_o,
                 pos_v, src_v, dst_v, bas_v, sem):
    wid = lax.axis_index("s") * _NC + lax.axis_index("c")
    base = wid * _JPW
    pltpu.sync_copy(pos_h.at[pl.ds(base, _JPW)], pos_v)
    for q in range(_JPW // 16):
        pv = pos_v[pl.ds(q * 16, 16)]
        pos_v[pl.ds(q * 16, 16)] = jnp.minimum(
            jnp.maximum(pv, 0), C - 1)
    pltpu.sync_copy(src_h.at[pl.ds(base, _JPW)], src_v)
    pltpu.sync_copy(dst_h.at[pl.ds(base, _JPW)], dst_v)
    pltpu.sync_copy(bas_h.at[pl.ds(base, _JPW)], bas_v)
    a = pltpu.async_copy(src_v, src_o.at[pos_v], sem)
    b = pltpu.async_copy(dst_v, dst_o.at[pos_v], sem)
    c = pltpu.async_copy(bas_v, bas_o.at[pos_v], sem)
    a.wait()
    b.wait()
    c.wait()


@functools.cache
def _make_sc_gather():
    @functools.partial(
        pl.kernel, mesh=_smesh(),
        out_type=jax.ShapeDtypeStruct((C, KPAD), jnp.float32),
        scratch_types=[
            pltpu.VMEM((_CH,), jnp.int32),
            pltpu.VMEM((_CH, KPAD), jnp.float32),
            pltpu.SemaphoreType.DMA,
        ],
    )
    def _sc_gather(h_h, src_h, out_h, idx_v, buf_v, sem):
        wid = lax.axis_index("s") * _NC + lax.axis_index("c")
        base = wid * _JPW
        for ch in range(4):
            off = base + ch * _CH
            pltpu.sync_copy(src_h.at[pl.ds(off, _CH)], idx_v)
            for q in range(_CH // 16):
                idx_v[pl.ds(q * 16, 16)] = (
                    idx_v[pl.ds(q * 16, 16)] & (N - 1))
            pltpu.async_copy(h_h.at[idx_v], buf_v, sem).wait()
            pltpu.sync_copy(buf_v, out_h.at[pl.ds(off, _CH)])

    return _sc_gather


@functools.cache
def _make_sc_scatadd():
    return functools.partial(
        pl.kernel, mesh=_smesh(),
        out_type=jax.ShapeDtypeStruct((2, N, KPAD), jnp.float32),
        scratch_types=[
            pltpu.VMEM_SHARED((N, KPAD), jnp.float32),
            pltpu.VMEM((_CH, KPAD), jnp.float32),
            pltpu.VMEM((_CH,), jnp.int32),
            pltpu.SemaphoreType.DMA,
        ],
    )(_sc_scatadd_body)


def _sc_scatadd_body(y_h, dst_h, zeros_h, out_h, acc_sh, ybuf, idxv, sem):
    cid = lax.axis_index("c")
    sid = lax.axis_index("s")
    wid = sid * _NC + cid
    pltpu.sync_copy(zeros_h.at[pl.ds(sid * _NPT, _NPT)],
                    acc_sh.at[pl.ds(sid * _NPT, _NPT)])
    plsc.subcore_barrier()
    base = wid * _JPW
    for ch in range(4):
        off = base + ch * _CH
        pltpu.sync_copy(dst_h.at[pl.ds(off, _CH)], idxv)
        for q in range(_CH // 16):
            idxv[pl.ds(q * 16, 16)] = idxv[pl.ds(q * 16, 16)] & (N - 1)
        pltpu.sync_copy(y_h.at[pl.ds(off, _CH)], ybuf)
        pltpu.sync_copy(ybuf, acc_sh.at[idxv], add=True)
    plsc.subcore_barrier()
    pltpu.sync_copy(acc_sh.at[pl.ds(sid * _NPT, _NPT)],
                    out_h.at[cid, pl.ds(sid * _NPT, _NPT)])


def _mm_body(in_ch, tk_ref, vals_ref, bas_ref, W_ref, y_ref):
    t = pl.program_id(0)
    k = tk_ref[t]
    A = vals_ref[:, :in_ch] * bas_ref[...]
    w = W_ref[pl.ds(k * in_ch, in_ch), :]
    yv = jnp.dot(A, w, preferred_element_type=jnp.float32)
    y_ref[...] = jnp.concatenate(
        [yv, jnp.zeros((128, KPAD - F), jnp.float32)], axis=1)


def _combine_body(p_ref, b_ref, o_ref):
    o_ref[...] = jnp.maximum(p_ref[0] + p_ref[1] + b_ref[...], 0.0)


def _final_body(x_ref, h1, h2, h3, h4, h5, h6,
                lw0, lw1, lw2, lw3, lw4, lw5, lw6, lb_ref, o_ref):
    hs = [x_ref, h1, h2, h3, h4, h5, h6]
    lws = [lw0, lw1, lw2, lw3, lw4, lw5, lw6]
    acc = jnp.zeros((F, N), jnp.float32)
    for h_ref, lw_ref in zip(hs, lws):
        acc += lax.dot_general(lw_ref[...], h_ref[...],
                               (((0,), (1,)), ((), ())),
                               preferred_element_type=jnp.float32)
    o_ref[...] = acc + lb_ref[...]


def kernel(x, edge_index, edge_attr, w0, root0, b0, w1, root1, b1,
           w2, root2, b2, w3, root3, b3, w4, root4, b4, w5, root5, b5,
           lin_w, lin_b):
    f32 = jnp.float32
    i32 = jnp.int32
    x_pad = jnp.pad(x.astype(f32), ((0, 0), (0, KPAD - x.shape[1])))
    attr_t = edge_attr.astype(f32).T
    ei = edge_index.astype(i32)

    def flat_w(w, r, in_p):
        kp = jnp.zeros((KPAD, in_p, F), f32)
        kp = kp.at[:K, :w.shape[1], :].set(w.astype(f32))
        kp = kp.at[K, :r.shape[0], :].set(r.astype(f32))
        return kp.reshape(KPAD * in_p, F)

    Ws = [flat_w(w0, root0, IN0P)] + [
        flat_w(w, r, F) for w, r in
        ((w1, root1), (w2, root2), (w3, root3), (w4, root4), (w5, root5))]
    Bs = [jnp.pad(b.astype(f32).reshape(1, F), ((0, 0), (0, KPAD - F)))
          for b in (b0, b1, b2, b3, b4, b5)]

    lw0p = jnp.zeros((KPAD, F), f32).at[:x.shape[1], :].set(
        lin_w[:x.shape[1], :].astype(f32))
    lws = [lw0p] + [
        jnp.zeros((KPAD, F), f32).at[:F, :].set(
            lin_w[x.shape[1] + l * F:x.shape[1] + (l + 1) * F,
                  :].astype(f32)) for l in range(LAYERS)]
    lb_col = lin_b.astype(f32).reshape(F, 1)

    ts2048 = jnp.tril(jnp.ones((JBLK, JBLK), f32), -1)
    ts128 = jnp.triu(jnp.ones((KPAD, KPAD), f32), 1)
    zeros_nf = jnp.zeros((N, KPAD), f32)

    neb = E // EBLK
    basT, widxT, srep, drep, deg = pl.pallas_call(
        _prep1_body,
        grid=(neb,),
        in_specs=[pl.BlockSpec((2, EBLK), lambda i: (0, i)),
                  pl.BlockSpec((3, EBLK), lambda i: (0, i))],
        out_specs=[pl.BlockSpec((EBLK, 8), lambda i: (i, 0)),
                   pl.BlockSpec((EBLK, 8), lambda i: (i, 0)),
                   pl.BlockSpec((EBLK, 8), lambda i: (i, 0)),
                   pl.BlockSpec((EBLK, 8), lambda i: (i, 0)),
                   pl.BlockSpec((N, 1), lambda i: (0, 0))],
        out_shape=[jax.ShapeDtypeStruct((E, 8), f32),
                   jax.ShapeDtypeStruct((E, 8), i32),
                   jax.ShapeDtypeStruct((E, 8), i32),
                   jax.ShapeDtypeStruct((E, 8), i32),
                   jax.ShapeDtypeStruct((N, 1), f32)],
        compiler_params=_VM,
    )(ei, attr_t)

    basf = pl.pallas_call(
        _prep1b_body,
        grid=(neb,),
        in_specs=[pl.BlockSpec((2, EBLK), lambda i: (0, i)),
                  pl.BlockSpec((N, 1), lambda i: (0, 0)),
                  pl.BlockSpec((EBLK, 8), lambda i: (i, 0))],
        out_specs=pl.BlockSpec((EBLK, 8), lambda i: (i, 0)),
        out_shape=jax.ShapeDtypeStruct((E, 8), f32),
        compiler_params=_VM,
    )(ei, deg, basT)

    key_col = widxT.reshape(J, 1)
    rank, cnt3 = pl.pallas_call(
        _prep2_body,
        grid=(NJB,),
        in_specs=[pl.BlockSpec((JBLK, 1), lambda i: (i, 0)),
                  pl.BlockSpec((JBLK, JBLK), lambda i: (0, 0))],
        out_specs=[pl.BlockSpec((JBLK, 1), lambda i: (i, 0)),
                   pl.BlockSpec((1, 1, KPAD), lambda i: (i, 0, 0))],
        out_shape=[jax.ShapeDtypeStruct((J, 1), f32),
                   jax.ShapeDtypeStruct((NJB, 1, KPAD), f32)],
        compiler_params=_VM,
    )(key_col, ts2048)

    soff, tkmap, pos_pad = pl.pallas_call(
        _prep4_body,
        in_specs=[pl.BlockSpec((NJB, 1, KPAD), lambda: (0, 0, 0)),
                  pl.BlockSpec((KPAD, KPAD), lambda: (0, 0))],
        out_specs=[pl.BlockSpec((1, KPAD), lambda: (0, 0)),
                   pl.BlockSpec((T_TOT + 112, 1), lambda: (0, 0)),
                   pl.BlockSpec((NPAD, 1), lambda: (0, 0))],
        out_shape=[jax.ShapeDtypeStruct((1, KPAD), f32),
                   jax.ShapeDtypeStruct((T_TOT + 112, 1), i32),
                   jax.ShapeDtypeStruct((NPAD, 1), i32)],
        compiler_params=_VM,
    )(cnt3, ts128)

    pos_main = pl.pallas_call(
        _prep3_body,
        grid=(NJB,),
        in_specs=[pl.BlockSpec((JBLK, 1), lambda i: (i, 0)),
                  pl.BlockSpec((JBLK, 1), lambda i: (i, 0)),
                  pl.BlockSpec((NJB, 1, KPAD), lambda i: (0, 0, 0)),
                  pl.BlockSpec((1, KPAD), lambda i: (0, 0))],
        out_specs=pl.BlockSpec((JBLK, 1), lambda i: (i, 0)),
        out_shape=jax.ShapeDtypeStruct((J, 1), i32),
        compiler_params=_VM,
    )(key_col, rank, cnt3, soff)

    sd_pad = (jnp.arange(NPAD, dtype=i32) & (N - 1))
    ar_n = jnp.arange(N, dtype=i32)
    pos_ext = jnp.concatenate(
        [pos_main.reshape(J), pos_pad.reshape(NPAD), ar_n])
    src_ext = jnp.concatenate([srep.reshape(J), sd_pad, ar_n])
    dst_ext = jnp.concatenate([drep.reshape(J), sd_pad, ar_n])
    bas_ext = jnp.concatenate(
        [basf.reshape(J), jnp.zeros((NPAD,), f32), jnp.ones((N,), f32)])

    src_s, dst_s, bas_s = _make_sc_sortscat()(pos_ext, src_ext, dst_ext, bas_ext)
    bas_col = bas_s.reshape(C, 1)
    tk = tkmap.reshape(T_TOT + 112)[:T_TOT]

    h = x_pad
    feats = []
    for l in range(LAYERS):
        in_ch = IN0P if l == 0 else F
        vals = _make_sc_gather()(h, src_s)
        y = pl.pallas_call(
            functools.partial(_mm_body, in_ch),
            grid_spec=pltpu.PrefetchScalarGridSpec(
                num_scalar_prefetch=1,
                grid=(T_TOT,),
                in_specs=[
                    pl.BlockSpec((128, KPAD), lambda t, tk_: (t, 0)),
                    pl.BlockSpec((128, 1), lambda t, tk_: (t, 0)),
                    pl.BlockSpec((KPAD * in_ch, F), lambda t, tk_: (0, 0)),
                ],
                out_specs=pl.BlockSpec((128, KPAD), lambda t, tk_: (t, 0)),
            ),
            out_shape=jax.ShapeDtypeStruct((C, KPAD), f32),
            compiler_params=_VM,
        )(tk, vals, bas_col, Ws[l])
        parts = _make_sc_scatadd()(y, dst_s, zeros_nf)
        h = pl.pallas_call(
            _combine_body,
            in_specs=[pl.BlockSpec((2, N, KPAD), lambda: (0, 0, 0)),
                      pl.BlockSpec((1, KPAD), lambda: (0, 0))],
            out_specs=pl.BlockSpec((N, KPAD), lambda: (0, 0)),
            out_shape=jax.ShapeDtypeStruct((N, KPAD), f32),
            compiler_params=_VM,
        )(parts, Bs[l])
        feats.append(h)

    out = pl.pallas_call(
        _final_body,
        in_specs=[pl.BlockSpec((N, KPAD), lambda: (0, 0))] * (LAYERS + 1)
        + [pl.BlockSpec((KPAD, F), lambda: (0, 0))] * (LAYERS + 1)
        + [pl.BlockSpec((F, 1), lambda: (0, 0))],
        out_specs=pl.BlockSpec((F, N), lambda: (0, 0)),
        out_shape=jax.ShapeDtypeStruct((F, N), f32),
        compiler_params=_VM,
    )(x_pad, *feats, *lws, lb_col)
    return out


# speedup vs baseline: 2.7195x; 2.7195x over previous
import functools

import jax
import jax.numpy as jnp
from jax import lax
from jax.experimental import pallas as pl
from jax.experimental.pallas import tpu as pltpu
from jax.experimental.pallas import tpu_sc as plsc

KS = 5
DIM = 3
K = 125
KPAD = 128
N = 2048
E = 8192
J = E * 8
F = 64
IN0P = 16
LAYERS = 6

RT_TILES = 16
DATA_TILES = 640
T_TOT = RT_TILES + DATA_TILES
C = T_TOT * 128
NPAD = 16384
EBLK = 1024
JBLK = 2048
NJB = J // JBLK

_NC = 2
_NS = 16
_NW = _NC * _NS
_JPW = C // _NW
_CH = _JPW // 4
_NPT = N // _NS

_VM = pltpu.CompilerParams(vmem_limit_bytes=100 * 1024 * 1024)


def _prep1_body(ei_ref, attr_ref, bas_ref, widx_ref, srep_ref, drep_ref,
                deg_ref):
    i = pl.program_id(0)
    eiT = jnp.transpose(ei_ref[...])
    srcc = eiT[:, 0:1]
    dstc = eiT[:, 1:2]
    dst_row = ei_ref[1:2, :]

    p = attr_ref[...] * (KS - 1.0)
    lo = jnp.floor(p)
    frac = p - lo
    lo_i = jnp.clip(lo.astype(jnp.int32), 0, KS - 1)

    bidx = lax.broadcasted_iota(jnp.int32, (8, 1), 0)
    basis8 = jnp.ones((8, EBLK), jnp.float32)
    widx8 = jnp.zeros((8, EBLK), jnp.int32)
    for d in range(DIM):
        bi = (bidx >> d) & 1
        bf = bi.astype(jnp.float32)
        f = frac[d:d + 1, :]
        basis8 = basis8 * (bf * f + (1.0 - bf) * (1.0 - f))
        ii = jnp.clip(lo_i[d:d + 1, :] + bi, 0, KS - 1)
        widx8 = widx8 * KS + ii

    bas_ref[...] = jnp.transpose(basis8)
    widx_ref[...] = jnp.transpose(widx8)
    srep_ref[...] = jnp.broadcast_to(srcc, (EBLK, 8))
    drep_ref[...] = jnp.broadcast_to(dstc, (EBLK, 8))

    nio0 = lax.broadcasted_iota(jnp.int32, (N, EBLK), 0)
    part = jnp.sum((dst_row == nio0).astype(jnp.float32), axis=1,
                   keepdims=True)

    @pl.when(i == 0)
    def _():
        deg_ref[...] = jnp.zeros_like(deg_ref)

    deg_ref[...] += part


def _prep1b_body(ei_ref, deg_ref, bas_ref, out_ref):
    eiT = jnp.transpose(ei_ref[...])
    dstc = eiT[:, 1:2]
    nio = lax.broadcasted_iota(jnp.int32, (EBLK, N), 1)
    Dblk = (dstc == nio).astype(jnp.float32)
    invd = 1.0 / jnp.maximum(deg_ref[...], 1.0)
    hi = invd.astype(jnp.bfloat16).astype(jnp.float32)
    lo = invd - hi
    inv_e = (jnp.dot(Dblk, hi, preferred_element_type=jnp.float32)
             + jnp.dot(Dblk, lo, preferred_element_type=jnp.float32))
    out_ref[...] = bas_ref[...] * inv_e


def _prep2_body(key_ref, ts_ref, rank_ref, cnt_ref):
    kio = lax.broadcasted_iota(jnp.int32, (JBLK, KPAD), 1)
    oh = (key_ref[...] == kio).astype(jnp.float32)
    tsoh = jnp.dot(ts_ref[...], oh, preferred_element_type=jnp.float32)
    rank_ref[...] = jnp.sum(tsoh * oh, axis=1, keepdims=True)
    cnt_ref[...] = jnp.sum(oh, axis=0, keepdims=True)[None]


def _prep4_body(cnt_ref, ts128_ref, soff_ref, tk_ref, ppad_ref):
    lane = lax.broadcasted_iota(jnp.int32, (1, KPAD), 1)
    total = jnp.sum(cnt_ref[...], axis=0)
    t_i = total.astype(jnp.int32)
    ntc = (t_i + 127) >> 7
    s123 = jnp.sum(jnp.where(lane <= 123, ntc, 0))
    nt = jnp.where(lane == 124, DATA_TILES - s123,
                   jnp.where(lane <= 123, ntc, 0))
    ntf = nt.astype(jnp.float32)

    def exact_prefix(v_i32):
        d0 = (v_i32 & 255).astype(jnp.float32)
        d1 = ((v_i32 >> 8) & 255).astype(jnp.float32)
        d2 = (v_i32 >> 16).astype(jnp.float32)
        p = jnp.dot(d0, ts128_ref[...], preferred_element_type=jnp.float32)
        p += 256.0 * jnp.dot(d1, ts128_ref[...],
                             preferred_element_type=jnp.float32)
        p += 65536.0 * jnp.dot(d2, ts128_ref[...],
                               preferred_element_type=jnp.float32)
        return p

    tsf = RT_TILES + exact_prefix(nt)
    soff = 128.0 * tsf
    soff_ref[...] = soff

    tio = lax.broadcasted_iota(jnp.int32, (T_TOT + 112, 1), 0)
    ends = tsf + ntf
    cntk = jnp.sum((ends <= tio.astype(jnp.float32)).astype(jnp.float32),
                   axis=1, keepdims=True)
    tk = jnp.where(tio < RT_TILES, K,
                   jnp.minimum(cntk.astype(jnp.int32), KPAD - 1))
    tk_ref[...] = tk

    padk = 128.0 * ntf - total * (lane <= 124)
    padst = exact_prefix(padk.astype(jnp.int32))
    padend = padst + padk
    cio = lax.broadcasted_iota(jnp.int32, (NPAD, 1), 0).astype(jnp.float32)
    kc = jnp.sum((padend <= cio).astype(jnp.float32), axis=1,
                 keepdims=True)
    kio2 = lax.broadcasted_iota(jnp.int32, (NPAD, KPAD), 1)
    ohc = (kc.astype(jnp.int32) == kio2).astype(jnp.float32)
    ps_c = jnp.sum(ohc * padst, axis=1, keepdims=True)
    cnt_c = jnp.sum(ohc * total, axis=1, keepdims=True)
    so_c = jnp.sum(ohc * soff, axis=1, keepdims=True)
    ppad_ref[...] = (so_c + cnt_c + (cio - ps_c)).astype(jnp.int32)


def _prep3_body(key_ref, rank_ref, cnt_ref, soff_ref, pos_ref):
    i = pl.program_id(0)
    kio = lax.broadcasted_iota(jnp.int32, (JBLK, KPAD), 1)
    oh = (key_ref[...] == kio).astype(jnp.float32)
    rmask = (lax.broadcasted_iota(jnp.int32, (NJB, 1, 1), 0)
             < i).astype(jnp.float32)
    bp = jnp.sum(cnt_ref[...] * rmask, axis=0)
    base = soff_ref[...] + bp
    pos = jnp.sum(oh * base, axis=1, keepdims=True) + rank_ref[...]
    pos_ref[...] = pos.astype(jnp.int32)


@functools.cache
def _smesh():
    return plsc.VectorSubcoreMesh(core_axis_name="c", subcore_axis_name="s")


@functools.cache
def _make_sc_sortscat():
    jpt = C // _NS
    opw = C // _NW

    @functools.partial(
        pl.kernel, mesh=_smesh(),
        out_type=[
            jax.ShapeDtypeStruct((C,), jnp.int32),
            jax.ShapeDtypeStruct((C,), jnp.int32),
            jax.ShapeDtypeStruct((C,), jnp.float32),
        ],
        scratch_types=[
            pltpu.VMEM_SHARED((C,), jnp.int32),
            pltpu.VMEM_SHARED((C,), jnp.int32),
            pltpu.VMEM_SHARED((C,), jnp.float32),
            pltpu.VMEM((jpt,), jnp.int32),
            pltpu.VMEM((jpt,), jnp.int32),
            pltpu.VMEM((jpt,), jnp.int32),
            pltpu.VMEM((jpt,), jnp.float32),
            pltpu.VMEM((opw,), jnp.int32),
            pltpu.VMEM((opw,), jnp.float32),
            pltpu.SemaphoreType.DMA,
        ],
    )
    def _sc_sortscat(pos_h, src_h, dst_h, bas_h, src_o, dst_o, bas_o,
                     src_sp, dst_sp, bas_sp, pos_v, src_v, dst_v, bas_v,
                     bi_v, bf_v, sem):
        cid = lax.axis_index("c")
        sid = lax.axis_index("s")
        base = sid * jpt
        pltpu.sync_copy(pos_h.at[pl.ds(base, jpt)], pos_v)
        for q in range(jpt // 16):
            pv = pos_v[pl.ds(q * 16, 16)]
            pos_v[pl.ds(q * 16, 16)] = jnp.minimum(
                jnp.maximum(pv, 0), C - 1)
        pltpu.sync_copy(src_h.at[pl.ds(base, jpt)], src_v)
        pltpu.sync_copy(dst_h.at[pl.ds(base, jpt)], dst_v)
        pltpu.sync_copy(bas_h.at[pl.ds(base, jpt)], bas_v)
        pltpu.sync_copy(src_v, src_sp.at[pos_v])
        pltpu.sync_copy(dst_v, dst_sp.at[pos_v])
        pltpu.sync_copy(bas_v, bas_sp.at[pos_v])
        plsc.subcore_barrier()
        out0 = cid * (C // _NC) + sid * opw
        pltpu.sync_copy(src_sp.at[pl.ds(out0, opw)], bi_v)
        pltpu.sync_copy(bi_v, src_o.at[pl.ds(out0, opw)])
        pltpu.sync_copy(dst_sp.at[pl.ds(out0, opw)], bi_v)
        pltpu.sync_copy(bi_v, dst_o.at[pl.ds(out0, opw)])
        pltpu.sync_copy(bas_sp.at[pl.ds(out0, opw)], bf_v)
        pltpu.sync_copy(bf_v, bas_o.at[pl.ds(out0, opw)])

    return _sc_sortscat


@functools.cache
def _make_sc_gather():
    @functools.partial(
        pl.kernel, mesh=_smesh(),
        out_type=jax.ShapeDtypeStruct((C, KPAD), jnp.float32),
        scratch_types=[
            pltpu.VMEM((_CH,), jnp.int32),
            pltpu.VMEM((_CH, KPAD), jnp.float32),
            pltpu.SemaphoreType.DMA,
        ],
    )
    def _sc_gather(h_h, src_h, out_h, idx_v, buf_v, sem):
        wid = lax.axis_index("s") * _NC + lax.axis_index("c")
        base = wid * _JPW
        for ch in range(4):
            off = base + ch * _CH
            pltpu.sync_copy(src_h.at[pl.ds(off, _CH)], idx_v)
            for q in range(_CH // 16):
                idx_v[pl.ds(q * 16, 16)] = (
                    idx_v[pl.ds(q * 16, 16)] & (N - 1))
            pltpu.async_copy(h_h.at[idx_v], buf_v, sem).wait()
            pltpu.sync_copy(buf_v, out_h.at[pl.ds(off, _CH)])

    return _sc_gather


@functools.cache
def _make_sc_scatadd():
    return functools.partial(
        pl.kernel, mesh=_smesh(),
        out_type=jax.ShapeDtypeStruct((2, N, KPAD), jnp.float32),
        scratch_types=[
            pltpu.VMEM_SHARED((N, KPAD), jnp.float32),
            pltpu.VMEM((_CH, KPAD), jnp.float32),
            pltpu.VMEM((_CH,), jnp.int32),
            pltpu.SemaphoreType.DMA,
        ],
    )(_sc_scatadd_body)


def _sc_scatadd_body(y_h, dst_h, zeros_h, out_h, acc_sh, ybuf, idxv, sem):
    cid = lax.axis_index("c")
    sid = lax.axis_index("s")
    wid = sid * _NC + cid
    pltpu.sync_copy(zeros_h.at[pl.ds(sid * _NPT, _NPT)],
                    acc_sh.at[pl.ds(sid * _NPT, _NPT)])
    plsc.subcore_barrier()
    base = wid * _JPW
    for ch in range(4):
        off = base + ch * _CH
        pltpu.sync_copy(dst_h.at[pl.ds(off, _CH)], idxv)
        for q in range(_CH // 16):
            idxv[pl.ds(q * 16, 16)] = idxv[pl.ds(q * 16, 16)] & (N - 1)
        pltpu.sync_copy(y_h.at[pl.ds(off, _CH)], ybuf)
        pltpu.sync_copy(ybuf, acc_sh.at[idxv], add=True)
    plsc.subcore_barrier()
    pltpu.sync_copy(acc_sh.at[pl.ds(sid * _NPT, _NPT)],
                    out_h.at[cid, pl.ds(sid * _NPT, _NPT)])


def _mm_body(in_ch, tk_ref, vals_ref, bas_ref, W_ref, y_ref):
    t = pl.program_id(0)
    for j in range(8):
        k = tk_ref[t * 8 + j]
        A = (vals_ref[j * 128:(j + 1) * 128, :in_ch]
             * bas_ref[j * 128:(j + 1) * 128, :])
        w = W_ref[pl.ds(k * in_ch, in_ch), :]
        yv = jnp.dot(A, w, preferred_element_type=jnp.float32)
        y_ref[j * 128:(j + 1) * 128, :] = jnp.concatenate(
            [yv, jnp.zeros((128, KPAD - F), jnp.float32)], axis=1)


def _combine_body(p_ref, b_ref, o_ref):
    o_ref[...] = jnp.maximum(p_ref[0] + p_ref[1] + b_ref[...], 0.0)


def _final_body(x_ref, h1, h2, h3, h4, h5, h6,
                lw0, lw1, lw2, lw3, lw4, lw5, lw6, lb_ref, o_ref):
    hs = [x_ref, h1, h2, h3, h4, h5, h6]
    lws = [lw0, lw1, lw2, lw3, lw4, lw5, lw6]
    acc = jnp.zeros((F, N), jnp.float32)
    for h_ref, lw_ref in zip(hs, lws):
        acc += lax.dot_general(lw_ref[...], h_ref[...],
                               (((0,), (1,)), ((), ())),
                               preferred_element_type=jnp.float32)
    o_ref[...] = acc + lb_ref[...]


def kernel(x, edge_index, edge_attr, w0, root0, b0, w1, root1, b1,
           w2, root2, b2, w3, root3, b3, w4, root4, b4, w5, root5, b5,
           lin_w, lin_b):
    f32 = jnp.float32
    i32 = jnp.int32
    x_pad = jnp.pad(x.astype(f32), ((0, 0), (0, KPAD - x.shape[1])))
    attr_t = edge_attr.astype(f32).T
    ei = edge_index.astype(i32)

    def flat_w(w, r, in_p):
        kp = jnp.zeros((KPAD, in_p, F), f32)
        kp = kp.at[:K, :w.shape[1], :].set(w.astype(f32))
        kp = kp.at[K, :r.shape[0], :].set(r.astype(f32))
        return kp.reshape(KPAD * in_p, F)

    Ws = [flat_w(w0, root0, IN0P)] + [
        flat_w(w, r, F) for w, r in
        ((w1, root1), (w2, root2), (w3, root3), (w4, root4), (w5, root5))]
    Bs = [jnp.pad(b.astype(f32).reshape(1, F), ((0, 0), (0, KPAD - F)))
          for b in (b0, b1, b2, b3, b4, b5)]

    lw0p = jnp.zeros((KPAD, F), f32).at[:x.shape[1], :].set(
        lin_w[:x.shape[1], :].astype(f32))
    lws = [lw0p] + [
        jnp.zeros((KPAD, F), f32).at[:F, :].set(
            lin_w[x.shape[1] + l * F:x.shape[1] + (l + 1) * F,
                  :].astype(f32)) for l in range(LAYERS)]
    lb_col = lin_b.astype(f32).reshape(F, 1)

    ts2048 = jnp.tril(jnp.ones((JBLK, JBLK), f32), -1)
    ts128 = jnp.triu(jnp.ones((KPAD, KPAD), f32), 1)
    zeros_nf = jnp.zeros((N, KPAD), f32)

    neb = E // EBLK
    basT, widxT, srep, drep, deg = pl.pallas_call(
        _prep1_body,
        grid=(neb,),
        in_specs=[pl.BlockSpec((2, EBLK), lambda i: (0, i)),
                  pl.BlockSpec((3, EBLK), lambda i: (0, i))],
        out_specs=[pl.BlockSpec((EBLK, 8), lambda i: (i, 0)),
                   pl.BlockSpec((EBLK, 8), lambda i: (i, 0)),
                   pl.BlockSpec((EBLK, 8), lambda i: (i, 0)),
                   pl.BlockSpec((EBLK, 8), lambda i: (i, 0)),
                   pl.BlockSpec((N, 1), lambda i: (0, 0))],
        out_shape=[jax.ShapeDtypeStruct((E, 8), f32),
                   jax.ShapeDtypeStruct((E, 8), i32),
                   jax.ShapeDtypeStruct((E, 8), i32),
                   jax.ShapeDtypeStruct((E, 8), i32),
                   jax.ShapeDtypeStruct((N, 1), f32)],
        compiler_params=_VM,
    )(ei, attr_t)

    basf = pl.pallas_call(
        _prep1b_body,
        grid=(neb,),
        in_specs=[pl.BlockSpec((2, EBLK), lambda i: (0, i)),
                  pl.BlockSpec((N, 1), lambda i: (0, 0)),
                  pl.BlockSpec((EBLK, 8), lambda i: (i, 0))],
        out_specs=pl.BlockSpec((EBLK, 8), lambda i: (i, 0)),
        out_shape=jax.ShapeDtypeStruct((E, 8), f32),
        compiler_params=_VM,
    )(ei, deg, basT)

    key_col = widxT.reshape(J, 1)
    rank, cnt3 = pl.pallas_call(
        _prep2_body,
        grid=(NJB,),
        in_specs=[pl.BlockSpec((JBLK, 1), lambda i: (i, 0)),
                  pl.BlockSpec((JBLK, JBLK), lambda i: (0, 0))],
        out_specs=[pl.BlockSpec((JBLK, 1), lambda i: (i, 0)),
                   pl.BlockSpec((1, 1, KPAD), lambda i: (i, 0, 0))],
        out_shape=[jax.ShapeDtypeStruct((J, 1), f32),
                   jax.ShapeDtypeStruct((NJB, 1, KPAD), f32)],
        compiler_params=_VM,
    )(key_col, ts2048)

    soff, tkmap, pos_pad = pl.pallas_call(
        _prep4_body,
        in_specs=[pl.BlockSpec((NJB, 1, KPAD), lambda: (0, 0, 0)),
                  pl.BlockSpec((KPAD, KPAD), lambda: (0, 0))],
        out_specs=[pl.BlockSpec((1, KPAD), lambda: (0, 0)),
                   pl.BlockSpec((T_TOT + 112, 1), lambda: (0, 0)),
                   pl.BlockSpec((NPAD, 1), lambda: (0, 0))],
        out_shape=[jax.ShapeDtypeStruct((1, KPAD), f32),
                   jax.ShapeDtypeStruct((T_TOT + 112, 1), i32),
                   jax.ShapeDtypeStruct((NPAD, 1), i32)],
        compiler_params=_VM,
    )(cnt3, ts128)

    pos_main = pl.pallas_call(
        _prep3_body,
        grid=(NJB,),
        in_specs=[pl.BlockSpec((JBLK, 1), lambda i: (i, 0)),
                  pl.BlockSpec((JBLK, 1), lambda i: (i, 0)),
                  pl.BlockSpec((NJB, 1, KPAD), lambda i: (0, 0, 0)),
                  pl.BlockSpec((1, KPAD), lambda i: (0, 0))],
        out_specs=pl.BlockSpec((JBLK, 1), lambda i: (i, 0)),
        out_shape=jax.ShapeDtypeStruct((J, 1), i32),
        compiler_params=_VM,
    )(key_col, rank, cnt3, soff)

    sd_pad = (jnp.arange(NPAD, dtype=i32) & (N - 1))
    ar_n = jnp.arange(N, dtype=i32)
    pos_ext = jnp.concatenate(
        [pos_main.reshape(J), pos_pad.reshape(NPAD), ar_n])
    src_ext = jnp.concatenate([srep.reshape(J), sd_pad, ar_n])
    dst_ext = jnp.concatenate([drep.reshape(J), sd_pad, ar_n])
    bas_ext = jnp.concatenate(
        [basf.reshape(J), jnp.zeros((NPAD,), f32), jnp.ones((N,), f32)])

    src_s, dst_s, bas_s = _make_sc_sortscat()(pos_ext, src_ext, dst_ext, bas_ext)
    bas_col = bas_s.reshape(C, 1)
    tk = tkmap.reshape(T_TOT + 112)[:T_TOT]

    h = x_pad
    feats = []
    for l in range(LAYERS):
        in_ch = IN0P if l == 0 else F
        vals = _make_sc_gather()(h, src_s)
        y = pl.pallas_call(
            functools.partial(_mm_body, in_ch),
            grid_spec=pltpu.PrefetchScalarGridSpec(
                num_scalar_prefetch=1,
                grid=(T_TOT // 8,),
                in_specs=[
                    pl.BlockSpec((1024, KPAD), lambda t, tk_: (t, 0)),
                    pl.BlockSpec((1024, 1), lambda t, tk_: (t, 0)),
                    pl.BlockSpec((KPAD * in_ch, F), lambda t, tk_: (0, 0)),
                ],
                out_specs=pl.BlockSpec((1024, KPAD), lambda t, tk_: (t, 0)),
            ),
            out_shape=jax.ShapeDtypeStruct((C, KPAD), f32),
            compiler_params=_VM,
        )(tk, vals, bas_col, Ws[l])
        parts = _make_sc_scatadd()(y, dst_s, zeros_nf)
        h = pl.pallas_call(
            _combine_body,
            in_specs=[pl.BlockSpec((2, N, KPAD), lambda: (0, 0, 0)),
                      pl.BlockSpec((1, KPAD), lambda: (0, 0))],
            out_specs=pl.BlockSpec((N, KPAD), lambda: (0, 0)),
            out_shape=jax.ShapeDtypeStruct((N, KPAD), f32),
            compiler_params=_VM,
        )(parts, Bs[l])
        feats.append(h)

    out = pl.pallas_call(
        _final_body,
        in_specs=[pl.BlockSpec((N, KPAD), lambda: (0, 0))] * (LAYERS + 1)
        + [pl.BlockSpec((KPAD, F), lambda: (0, 0))] * (LAYERS + 1)
        + [pl.BlockSpec((F, 1), lambda: (0, 0))],
        out_specs=pl.BlockSpec((F, N), lambda: (0, 0)),
        out_shape=jax.ShapeDtypeStruct((F, N), f32),
        compiler_params=_VM,
    )(x_pad, *feats, *lws, lb_col)
    return out


# speedup vs baseline: 2.7845x; 1.0239x over previous
import functools

import jax
import jax.numpy as jnp
from jax import lax
from jax.experimental import pallas as pl
from jax.experimental.pallas import tpu as pltpu
from jax.experimental.pallas import tpu_sc as plsc

KS = 5
DIM = 3
K = 125
KPAD = 128
N = 2048
E = 8192
J = E * 8
F = 64
IN0P = 16
LAYERS = 6

RT_TILES = 16
DATA_TILES = 640
T_TOT = RT_TILES + DATA_TILES
C = T_TOT * 128
NPAD = 16384
EBLK = 1024
JBLK = 2048
NJB = J // JBLK

_NC = 2
_NS = 16
_NW = _NC * _NS
_JPW = C // _NW
_CH = _JPW // 8
_NPT = N // _NS

_VM = pltpu.CompilerParams(vmem_limit_bytes=100 * 1024 * 1024)


def _prep1_body(ei_ref, attr_ref, bas_ref, widx_ref, srep_ref, drep_ref,
                deg_ref):
    i = pl.program_id(0)
    eiT = jnp.transpose(ei_ref[...])
    srcc = eiT[:, 0:1]
    dstc = eiT[:, 1:2]
    dst_row = ei_ref[1:2, :]

    p = attr_ref[...] * (KS - 1.0)
    lo = jnp.floor(p)
    frac = p - lo
    lo_i = jnp.clip(lo.astype(jnp.int32), 0, KS - 1)

    bidx = lax.broadcasted_iota(jnp.int32, (8, 1), 0)
    basis8 = jnp.ones((8, EBLK), jnp.float32)
    widx8 = jnp.zeros((8, EBLK), jnp.int32)
    for d in range(DIM):
        bi = (bidx >> d) & 1
        bf = bi.astype(jnp.float32)
        f = frac[d:d + 1, :]
        basis8 = basis8 * (bf * f + (1.0 - bf) * (1.0 - f))
        ii = jnp.clip(lo_i[d:d + 1, :] + bi, 0, KS - 1)
        widx8 = widx8 * KS + ii

    bas_ref[...] = jnp.transpose(basis8)
    widx_ref[...] = jnp.transpose(widx8)
    srep_ref[...] = jnp.broadcast_to(srcc, (EBLK, 8))
    drep_ref[...] = jnp.broadcast_to(dstc, (EBLK, 8))

    nio0 = lax.broadcasted_iota(jnp.int32, (N, EBLK), 0)
    part = jnp.sum((dst_row == nio0).astype(jnp.float32), axis=1,
                   keepdims=True)

    @pl.when(i == 0)
    def _():
        deg_ref[...] = jnp.zeros_like(deg_ref)

    deg_ref[...] += part


def _prep1b_body(ei_ref, deg_ref, bas_ref, out_ref):
    eiT = jnp.transpose(ei_ref[...])
    dstc = eiT[:, 1:2]
    nio = lax.broadcasted_iota(jnp.int32, (EBLK, N), 1)
    Dblk = (dstc == nio).astype(jnp.float32)
    invd = 1.0 / jnp.maximum(deg_ref[...], 1.0)
    hi = invd.astype(jnp.bfloat16).astype(jnp.float32)
    lo = invd - hi
    inv_e = (jnp.dot(Dblk, hi, preferred_element_type=jnp.float32)
             + jnp.dot(Dblk, lo, preferred_element_type=jnp.float32))
    out_ref[...] = bas_ref[...] * inv_e


def _prep2_body(key_ref, ts_ref, rank_ref, cnt_ref):
    kio = lax.broadcasted_iota(jnp.int32, (JBLK, KPAD), 1)
    oh = (key_ref[...] == kio).astype(jnp.float32)
    tsoh = jnp.dot(ts_ref[...], oh, preferred_element_type=jnp.float32)
    rank_ref[...] = jnp.sum(tsoh * oh, axis=1, keepdims=True)
    cnt_ref[...] = jnp.sum(oh, axis=0, keepdims=True)[None]


def _prep4_body(cnt_ref, ts128_ref, soff_ref, tk_ref, ppad_ref):
    lane = lax.broadcasted_iota(jnp.int32, (1, KPAD), 1)
    total = jnp.sum(cnt_ref[...], axis=0)
    t_i = total.astype(jnp.int32)
    ntc = (t_i + 127) >> 7
    s123 = jnp.sum(jnp.where(lane <= 123, ntc, 0))
    nt = jnp.where(lane == 124, DATA_TILES - s123,
                   jnp.where(lane <= 123, ntc, 0))
    ntf = nt.astype(jnp.float32)

    def exact_prefix(v_i32):
        d0 = (v_i32 & 255).astype(jnp.float32)
        d1 = ((v_i32 >> 8) & 255).astype(jnp.float32)
        d2 = (v_i32 >> 16).astype(jnp.float32)
        p = jnp.dot(d0, ts128_ref[...], preferred_element_type=jnp.float32)
        p += 256.0 * jnp.dot(d1, ts128_ref[...],
                             preferred_element_type=jnp.float32)
        p += 65536.0 * jnp.dot(d2, ts128_ref[...],
                               preferred_element_type=jnp.float32)
        return p

    tsf = RT_TILES + exact_prefix(nt)
    soff = 128.0 * tsf
    soff_ref[...] = soff

    tio = lax.broadcasted_iota(jnp.int32, (T_TOT + 112, 1), 0)
    ends = tsf + ntf
    cntk = jnp.sum((ends <= tio.astype(jnp.float32)).astype(jnp.float32),
                   axis=1, keepdims=True)
    tk = jnp.where(tio < RT_TILES, K,
                   jnp.minimum(cntk.astype(jnp.int32), KPAD - 1))
    tk_ref[...] = tk

    padk = 128.0 * ntf - total * (lane <= 124)
    padst = exact_prefix(padk.astype(jnp.int32))
    padend = padst + padk
    cio = lax.broadcasted_iota(jnp.int32, (NPAD, 1), 0).astype(jnp.float32)
    kc = jnp.sum((padend <= cio).astype(jnp.float32), axis=1,
                 keepdims=True)
    kio2 = lax.broadcasted_iota(jnp.int32, (NPAD, KPAD), 1)
    ohc = (kc.astype(jnp.int32) == kio2).astype(jnp.float32)
    ps_c = jnp.sum(ohc * padst, axis=1, keepdims=True)
    cnt_c = jnp.sum(ohc * total, axis=1, keepdims=True)
    so_c = jnp.sum(ohc * soff, axis=1, keepdims=True)
    ppad_ref[...] = (so_c + cnt_c + (cio - ps_c)).astype(jnp.int32)


def _prep3_body(key_ref, rank_ref, cnt_ref, soff_ref, pos_ref):
    i = pl.program_id(0)
    kio = lax.broadcasted_iota(jnp.int32, (JBLK, KPAD), 1)
    oh = (key_ref[...] == kio).astype(jnp.float32)
    rmask = (lax.broadcasted_iota(jnp.int32, (NJB, 1, 1), 0)
             < i).astype(jnp.float32)
    bp = jnp.sum(cnt_ref[...] * rmask, axis=0)
    base = soff_ref[...] + bp
    pos = jnp.sum(oh * base, axis=1, keepdims=True) + rank_ref[...]
    pos_ref[...] = pos.astype(jnp.int32)


@functools.cache
def _smesh():
    return plsc.VectorSubcoreMesh(core_axis_name="c", subcore_axis_name="s")


@functools.cache
def _make_sc_sortscat():
    jpt = C // _NS
    opw = C // _NW

    @functools.partial(
        pl.kernel, mesh=_smesh(),
        out_type=[
            jax.ShapeDtypeStruct((C,), jnp.int32),
            jax.ShapeDtypeStruct((C,), jnp.int32),
            jax.ShapeDtypeStruct((C,), jnp.float32),
        ],
        scratch_types=[
            pltpu.VMEM_SHARED((C,), jnp.int32),
            pltpu.VMEM_SHARED((C,), jnp.int32),
            pltpu.VMEM_SHARED((C,), jnp.float32),
            pltpu.VMEM((jpt,), jnp.int32),
            pltpu.VMEM((jpt,), jnp.int32),
            pltpu.VMEM((jpt,), jnp.int32),
            pltpu.VMEM((jpt,), jnp.float32),
            pltpu.VMEM((opw,), jnp.int32),
            pltpu.VMEM((opw,), jnp.float32),
            pltpu.SemaphoreType.DMA,
        ],
    )
    def _sc_sortscat(pos_h, src_h, dst_h, bas_h, src_o, dst_o, bas_o,
                     src_sp, dst_sp, bas_sp, pos_v, src_v, dst_v, bas_v,
                     bi_v, bf_v, sem):
        cid = lax.axis_index("c")
        sid = lax.axis_index("s")
        base = sid * jpt
        pltpu.sync_copy(pos_h.at[pl.ds(base, jpt)], pos_v)
        for q in range(jpt // 16):
            pv = pos_v[pl.ds(q * 16, 16)]
            pos_v[pl.ds(q * 16, 16)] = jnp.minimum(
                jnp.maximum(pv, 0), C - 1)
        pltpu.sync_copy(src_h.at[pl.ds(base, jpt)], src_v)
        pltpu.sync_copy(dst_h.at[pl.ds(base, jpt)], dst_v)
        pltpu.sync_copy(bas_h.at[pl.ds(base, jpt)], bas_v)
        pltpu.sync_copy(src_v, src_sp.at[pos_v])
        pltpu.sync_copy(dst_v, dst_sp.at[pos_v])
        pltpu.sync_copy(bas_v, bas_sp.at[pos_v])
        plsc.subcore_barrier()
        out0 = cid * (C // _NC) + sid * opw
        pltpu.sync_copy(src_sp.at[pl.ds(out0, opw)], bi_v)
        pltpu.sync_copy(bi_v, src_o.at[pl.ds(out0, opw)])
        pltpu.sync_copy(dst_sp.at[pl.ds(out0, opw)], bi_v)
        pltpu.sync_copy(bi_v, dst_o.at[pl.ds(out0, opw)])
        pltpu.sync_copy(bas_sp.at[pl.ds(out0, opw)], bf_v)
        pltpu.sync_copy(bf_v, bas_o.at[pl.ds(out0, opw)])

    return _sc_sortscat


@functools.cache
def _make_sc_gather():
    @functools.partial(
        pl.kernel, mesh=_smesh(),
        out_type=jax.ShapeDtypeStruct((C, KPAD), jnp.float32),
        scratch_types=[
            pltpu.VMEM((_CH,), jnp.int32),
            pltpu.VMEM((_CH, KPAD), jnp.float32),
            pltpu.VMEM((_CH, KPAD), jnp.float32),
            pltpu.SemaphoreType.DMA,
            pltpu.SemaphoreType.DMA,
        ],
    )
    def _sc_gather(h_h, src_h, out_h, idx_v, buf0, buf1, sem, sem2):
        wid = lax.axis_index("s") * _NC + lax.axis_index("c")
        base = wid * _JPW
        bufs = [buf0, buf1]
        wbs = [None] * 8
        for ch in range(8):
            off = base + ch * _CH
            b = bufs[ch % 2]
            pltpu.sync_copy(src_h.at[pl.ds(off, _CH)], idx_v)
            for q in range(_CH // 16):
                idx_v[pl.ds(q * 16, 16)] = (
                    idx_v[pl.ds(q * 16, 16)] & (N - 1))
            if ch >= 2:
                wbs[ch - 2].wait()
            pltpu.async_copy(h_h.at[idx_v], b, sem).wait()
            wbs[ch] = pltpu.async_copy(b, out_h.at[pl.ds(off, _CH)], sem2)
        wbs[6].wait()
        wbs[7].wait()

    return _sc_gather


@functools.cache
def _make_sc_scatadd():
    return functools.partial(
        pl.kernel, mesh=_smesh(),
        out_type=jax.ShapeDtypeStruct((2, N, KPAD), jnp.float32),
        scratch_types=[
            pltpu.VMEM_SHARED((N, KPAD), jnp.float32),
            pltpu.VMEM((_CH, KPAD), jnp.float32),
            pltpu.VMEM((_CH, KPAD), jnp.float32),
            pltpu.VMEM((_CH,), jnp.int32),
            pltpu.VMEM((_CH,), jnp.int32),
            pltpu.SemaphoreType.DMA,
            pltpu.SemaphoreType.DMA,
        ],
    )(_sc_scatadd_body)


def _sc_scatadd_body(y_h, dst_h, zeros_h, out_h, acc_sh, ybuf0, ybuf1,
                     idx0, idx1, sem, sem2):
    cid = lax.axis_index("c")
    sid = lax.axis_index("s")
    wid = sid * _NC + cid
    pltpu.sync_copy(zeros_h.at[pl.ds(sid * _NPT, _NPT)],
                    acc_sh.at[pl.ds(sid * _NPT, _NPT)])
    plsc.subcore_barrier()
    base = wid * _JPW
    ybufs = [ybuf0, ybuf1]
    idxs = [idx0, idx1]
    adds = [None] * 8
    for ch in range(8):
        off = base + ch * _CH
        b = ybufs[ch % 2]
        ix = idxs[ch % 2]
        pltpu.sync_copy(dst_h.at[pl.ds(off, _CH)], ix)
        for q in range(_CH // 16):
            ix[pl.ds(q * 16, 16)] = ix[pl.ds(q * 16, 16)] & (N - 1)
        if ch >= 2:
            adds[ch - 2].wait()
        pltpu.async_copy(y_h.at[pl.ds(off, _CH)], b, sem).wait()
        adds[ch] = pltpu.async_copy(b, acc_sh.at[ix], sem2, add=True)
    adds[6].wait()
    adds[7].wait()
    plsc.subcore_barrier()
    pltpu.sync_copy(acc_sh.at[pl.ds(sid * _NPT, _NPT)],
                    out_h.at[cid, pl.ds(sid * _NPT, _NPT)])


def _mm_body(in_ch, tk_ref, vals_ref, bas_ref, W_ref, y_ref):
    t = pl.program_id(0)
    for j in range(8):
        k = tk_ref[t * 8 + j]
        A = (vals_ref[j * 128:(j + 1) * 128, :in_ch]
             * bas_ref[j * 128:(j + 1) * 128, :])
        w = W_ref[pl.ds(k * in_ch, in_ch), :]
        yv = jnp.dot(A, w, preferred_element_type=jnp.float32)
        y_ref[j * 128:(j + 1) * 128, :] = jnp.concatenate(
            [yv, jnp.zeros((128, KPAD - F), jnp.float32)], axis=1)


def _combine_body(p_ref, b_ref, o_ref):
    o_ref[...] = jnp.maximum(p_ref[0] + p_ref[1] + b_ref[...], 0.0)


def _final_body(x_ref, h1, h2, h3, h4, h5, h6,
                lw0, lw1, lw2, lw3, lw4, lw5, lw6, lb_ref, o_ref):
    hs = [x_ref, h1, h2, h3, h4, h5, h6]
    lws = [lw0, lw1, lw2, lw3, lw4, lw5, lw6]
    acc = jnp.zeros((F, N), jnp.float32)
    for h_ref, lw_ref in zip(hs, lws):
        acc += lax.dot_general(lw_ref[...], h_ref[...],
                               (((0,), (1,)), ((), ())),
                               preferred_element_type=jnp.float32)
    o_ref[...] = acc + lb_ref[...]


def kernel(x, edge_index, edge_attr, w0, root0, b0, w1, root1, b1,
           w2, root2, b2, w3, root3, b3, w4, root4, b4, w5, root5, b5,
           lin_w, lin_b):
    f32 = jnp.float32
    i32 = jnp.int32
    x_pad = jnp.pad(x.astype(f32), ((0, 0), (0, KPAD - x.shape[1])))
    attr_t = edge_attr.astype(f32).T
    ei = edge_index.astype(i32)

    def flat_w(w, r, in_p):
        kp = jnp.zeros((KPAD, in_p, F), f32)
        kp = kp.at[:K, :w.shape[1], :].set(w.astype(f32))
        kp = kp.at[K, :r.shape[0], :].set(r.astype(f32))
        return kp.reshape(KPAD * in_p, F)

    Ws = [flat_w(w0, root0, IN0P)] + [
        flat_w(w, r, F) for w, r in
        ((w1, root1), (w2, root2), (w3, root3), (w4, root4), (w5, root5))]
    Bs = [jnp.pad(b.astype(f32).reshape(1, F), ((0, 0), (0, KPAD - F)))
          for b in (b0, b1, b2, b3, b4, b5)]

    lw0p = jnp.zeros((KPAD, F), f32).at[:x.shape[1], :].set(
        lin_w[:x.shape[1], :].astype(f32))
    lws = [lw0p] + [
        jnp.zeros((KPAD, F), f32).at[:F, :].set(
            lin_w[x.shape[1] + l * F:x.shape[1] + (l + 1) * F,
                  :].astype(f32)) for l in range(LAYERS)]
    lb_col = lin_b.astype(f32).reshape(F, 1)

    ts2048 = jnp.tril(jnp.ones((JBLK, JBLK), f32), -1)
    ts128 = jnp.triu(jnp.ones((KPAD, KPAD), f32), 1)
    zeros_nf = jnp.zeros((N, KPAD), f32)

    neb = E // EBLK
    basT, widxT, srep, drep, deg = pl.pallas_call(
        _prep1_body,
        grid=(neb,),
        in_specs=[pl.BlockSpec((2, EBLK), lambda i: (0, i)),
                  pl.BlockSpec((3, EBLK), lambda i: (0, i))],
        out_specs=[pl.BlockSpec((EBLK, 8), lambda i: (i, 0)),
                   pl.BlockSpec((EBLK, 8), lambda i: (i, 0)),
                   pl.BlockSpec((EBLK, 8), lambda i: (i, 0)),
                   pl.BlockSpec((EBLK, 8), lambda i: (i, 0)),
                   pl.BlockSpec((N, 1), lambda i: (0, 0))],
        out_shape=[jax.ShapeDtypeStruct((E, 8), f32),
                   jax.ShapeDtypeStruct((E, 8), i32),
                   jax.ShapeDtypeStruct((E, 8), i32),
                   jax.ShapeDtypeStruct((E, 8), i32),
                   jax.ShapeDtypeStruct((N, 1), f32)],
        compiler_params=_VM,
    )(ei, attr_t)

    basf = pl.pallas_call(
        _prep1b_body,
        grid=(neb,),
        in_specs=[pl.BlockSpec((2, EBLK), lambda i: (0, i)),
                  pl.BlockSpec((N, 1), lambda i: (0, 0)),
                  pl.BlockSpec((EBLK, 8), lambda i: (i, 0))],
        out_specs=pl.BlockSpec((EBLK, 8), lambda i: (i, 0)),
        out_shape=jax.ShapeDtypeStruct((E, 8), f32),
        compiler_params=_VM,
    )(ei, deg, basT)

    key_col = widxT.reshape(J, 1)
    rank, cnt3 = pl.pallas_call(
        _prep2_body,
        grid=(NJB,),
        in_specs=[pl.BlockSpec((JBLK, 1), lambda i: (i, 0)),
                  pl.BlockSpec((JBLK, JBLK), lambda i: (0, 0))],
        out_specs=[pl.BlockSpec((JBLK, 1), lambda i: (i, 0)),
                   pl.BlockSpec((1, 1, KPAD), lambda i: (i, 0, 0))],
        out_shape=[jax.ShapeDtypeStruct((J, 1), f32),
                   jax.ShapeDtypeStruct((NJB, 1, KPAD), f32)],
        compiler_params=_VM,
    )(key_col, ts2048)

    soff, tkmap, pos_pad = pl.pallas_call(
        _prep4_body,
        in_specs=[pl.BlockSpec((NJB, 1, KPAD), lambda: (0, 0, 0)),
                  pl.BlockSpec((KPAD, KPAD), lambda: (0, 0))],
        out_specs=[pl.BlockSpec((1, KPAD), lambda: (0, 0)),
                   pl.BlockSpec((T_TOT + 112, 1), lambda: (0, 0)),
                   pl.BlockSpec((NPAD, 1), lambda: (0, 0))],
        out_shape=[jax.ShapeDtypeStruct((1, KPAD), f32),
                   jax.ShapeDtypeStruct((T_TOT + 112, 1), i32),
                   jax.ShapeDtypeStruct((NPAD, 1), i32)],
        compiler_params=_VM,
    )(cnt3, ts128)

    pos_main = pl.pallas_call(
        _prep3_body,
        grid=(NJB,),
        in_specs=[pl.BlockSpec((JBLK, 1), lambda i: (i, 0)),
                  pl.BlockSpec((JBLK, 1), lambda i: (i, 0)),
                  pl.BlockSpec((NJB, 1, KPAD), lambda i: (0, 0, 0)),
                  pl.BlockSpec((1, KPAD), lambda i: (0, 0))],
        out_specs=pl.BlockSpec((JBLK, 1), lambda i: (i, 0)),
        out_shape=jax.ShapeDtypeStruct((J, 1), i32),
        compiler_params=_VM,
    )(key_col, rank, cnt3, soff)

    sd_pad = (jnp.arange(NPAD, dtype=i32) & (N - 1))
    ar_n = jnp.arange(N, dtype=i32)
    pos_ext = jnp.concatenate(
        [pos_main.reshape(J), pos_pad.reshape(NPAD), ar_n])
    src_ext = jnp.concatenate([srep.reshape(J), sd_pad, ar_n])
    dst_ext = jnp.concatenate([drep.reshape(J), sd_pad, ar_n])
    bas_ext = jnp.concatenate(
        [basf.reshape(J), jnp.zeros((NPAD,), f32), jnp.ones((N,), f32)])

    src_s, dst_s, bas_s = _make_sc_sortscat()(pos_ext, src_ext, dst_ext, bas_ext)
    bas_col = bas_s.reshape(C, 1)
    tk = tkmap.reshape(T_TOT + 112)[:T_TOT]

    h = x_pad
    feats = []
    for l in range(LAYERS):
        in_ch = IN0P if l == 0 else F
        vals = _make_sc_gather()(h, src_s)
        y = pl.pallas_call(
            functools.partial(_mm_body, in_ch),
            grid_spec=pltpu.PrefetchScalarGridSpec(
                num_scalar_prefetch=1,
                grid=(T_TOT // 8,),
                in_specs=[
                    pl.BlockSpec((1024, KPAD), lambda t, tk_: (t, 0)),
                    pl.BlockSpec((1024, 1), lambda t, tk_: (t, 0)),
                    pl.BlockSpec((KPAD * in_ch, F), lambda t, tk_: (0, 0)),
                ],
                out_specs=pl.BlockSpec((1024, KPAD), lambda t, tk_: (t, 0)),
            ),
            out_shape=jax.ShapeDtypeStruct((C, KPAD), f32),
            compiler_params=_VM,
        )(tk, vals, bas_col, Ws[l])
        parts = _make_sc_scatadd()(y, dst_s, zeros_nf)
        h = pl.pallas_call(
            _combine_body,
            in_specs=[pl.BlockSpec((2, N, KPAD), lambda: (0, 0, 0)),
                      pl.BlockSpec((1, KPAD), lambda: (0, 0))],
            out_specs=pl.BlockSpec((N, KPAD), lambda: (0, 0)),
            out_shape=jax.ShapeDtypeStruct((N, KPAD), f32),
            compiler_params=_VM,
        )(parts, Bs[l])
        feats.append(h)

    out = pl.pallas_call(
        _final_body,
        in_specs=[pl.BlockSpec((N, KPAD), lambda: (0, 0))] * (LAYERS + 1)
        + [pl.BlockSpec((KPAD, F), lambda: (0, 0))] * (LAYERS + 1)
        + [pl.BlockSpec((F, 1), lambda: (0, 0))],
        out_specs=pl.BlockSpec((F, N), lambda: (0, 0)),
        out_shape=jax.ShapeDtypeStruct((F, N), f32),
        compiler_params=_VM,
    )(x_pad, *feats, *lws, lb_col)
    return out


# speedup vs baseline: 2.7984x; 1.0050x over previous
import functools

import jax
import jax.numpy as jnp
from jax import lax
from jax.experimental import pallas as pl
from jax.experimental.pallas import tpu as pltpu
from jax.experimental.pallas import tpu_sc as plsc

KS = 5
DIM = 3
K = 125
KPAD = 128
N = 2048
E = 8192
J = E * 8
F = 64
IN0P = 16
LAYERS = 6

RT_TILES = 16
DATA_TILES = 640
T_TOT = RT_TILES + DATA_TILES
C = T_TOT * 128
NPAD = 16384
EBLK = 1024
JBLK = 2048
NJB = J // JBLK

_NC = 2
_NS = 16
_NW = _NC * _NS
_JPW = C // _NW
_CH = _JPW // 8
_NPT = N // _NS

_VM = pltpu.CompilerParams(vmem_limit_bytes=100 * 1024 * 1024)


def _prep1_body(ei_ref, attr_ref, bas_ref, widx_ref, srep_ref, drep_ref,
                deg_ref):
    i = pl.program_id(0)
    eiT = jnp.transpose(ei_ref[...])
    srcc = eiT[:, 0:1]
    dstc = eiT[:, 1:2]
    dst_row = ei_ref[1:2, :]

    p = attr_ref[...] * (KS - 1.0)
    lo = jnp.floor(p)
    frac = p - lo
    lo_i = jnp.clip(lo.astype(jnp.int32), 0, KS - 1)

    bidx = lax.broadcasted_iota(jnp.int32, (8, 1), 0)
    basis8 = jnp.ones((8, EBLK), jnp.float32)
    widx8 = jnp.zeros((8, EBLK), jnp.int32)
    for d in range(DIM):
        bi = (bidx >> d) & 1
        bf = bi.astype(jnp.float32)
        f = frac[d:d + 1, :]
        basis8 = basis8 * (bf * f + (1.0 - bf) * (1.0 - f))
        ii = jnp.clip(lo_i[d:d + 1, :] + bi, 0, KS - 1)
        widx8 = widx8 * KS + ii

    bas_ref[...] = jnp.transpose(basis8)
    widx_ref[...] = jnp.transpose(widx8)
    srep_ref[...] = jnp.broadcast_to(srcc, (EBLK, 8))
    drep_ref[...] = jnp.broadcast_to(dstc, (EBLK, 8))

    nio0 = lax.broadcasted_iota(jnp.int32, (N, EBLK), 0)
    part = jnp.sum((dst_row == nio0).astype(jnp.float32), axis=1,
                   keepdims=True)

    @pl.when(i == 0)
    def _():
        deg_ref[...] = jnp.zeros_like(deg_ref)

    deg_ref[...] += part


def _prep1b_body(ei_ref, deg_ref, bas_ref, out_ref):
    eiT = jnp.transpose(ei_ref[...])
    dstc = eiT[:, 1:2]
    nio = lax.broadcasted_iota(jnp.int32, (EBLK, N), 1)
    Dblk = (dstc == nio).astype(jnp.float32)
    invd = 1.0 / jnp.maximum(deg_ref[...], 1.0)
    hi = invd.astype(jnp.bfloat16).astype(jnp.float32)
    lo = invd - hi
    inv_e = (jnp.dot(Dblk, hi, preferred_element_type=jnp.float32)
             + jnp.dot(Dblk, lo, preferred_element_type=jnp.float32))
    out_ref[...] = bas_ref[...] * inv_e


def _prep2_body(key_ref, ts_ref, rank_ref, cnt_ref):
    kio = lax.broadcasted_iota(jnp.int32, (JBLK, KPAD), 1)
    oh = (key_ref[...] == kio).astype(jnp.float32)
    tsoh = jnp.dot(ts_ref[...], oh, preferred_element_type=jnp.float32)
    rank_ref[...] = jnp.sum(tsoh * oh, axis=1, keepdims=True)
    cnt_ref[...] = jnp.sum(oh, axis=0, keepdims=True)[None]


def _prep4_body(cnt_ref, ts128_ref, soff_ref, tk_ref, ppad_ref):
    lane = lax.broadcasted_iota(jnp.int32, (1, KPAD), 1)
    total = jnp.sum(cnt_ref[...], axis=0)
    t_i = total.astype(jnp.int32)
    ntc = (t_i + 127) >> 7
    s123 = jnp.sum(jnp.where(lane <= 123, ntc, 0))
    nt = jnp.where(lane == 124, DATA_TILES - s123,
                   jnp.where(lane <= 123, ntc, 0))
    ntf = nt.astype(jnp.float32)

    def exact_prefix(v_i32):
        d0 = (v_i32 & 255).astype(jnp.float32)
        d1 = ((v_i32 >> 8) & 255).astype(jnp.float32)
        d2 = (v_i32 >> 16).astype(jnp.float32)
        p = jnp.dot(d0, ts128_ref[...], preferred_element_type=jnp.float32)
        p += 256.0 * jnp.dot(d1, ts128_ref[...],
                             preferred_element_type=jnp.float32)
        p += 65536.0 * jnp.dot(d2, ts128_ref[...],
                               preferred_element_type=jnp.float32)
        return p

    tsf = RT_TILES + exact_prefix(nt)
    soff = 128.0 * tsf
    soff_ref[...] = soff

    tio = lax.broadcasted_iota(jnp.int32, (T_TOT + 112, 1), 0)
    ends = tsf + ntf
    cntk = jnp.sum((ends <= tio.astype(jnp.float32)).astype(jnp.float32),
                   axis=1, keepdims=True)
    tk = jnp.where(tio < RT_TILES, K,
                   jnp.minimum(cntk.astype(jnp.int32), KPAD - 1))
    tk_ref[...] = tk

    padk = 128.0 * ntf - total * (lane <= 124)
    padst = exact_prefix(padk.astype(jnp.int32))
    padend = padst + padk
    cio = lax.broadcasted_iota(jnp.int32, (NPAD, 1), 0).astype(jnp.float32)
    kc = jnp.sum((padend <= cio).astype(jnp.float32), axis=1,
                 keepdims=True)
    kio2 = lax.broadcasted_iota(jnp.int32, (NPAD, KPAD), 1)
    ohc = (kc.astype(jnp.int32) == kio2).astype(jnp.float32)
    ps_c = jnp.sum(ohc * padst, axis=1, keepdims=True)
    cnt_c = jnp.sum(ohc * total, axis=1, keepdims=True)
    so_c = jnp.sum(ohc * soff, axis=1, keepdims=True)
    ppad_ref[...] = (so_c + cnt_c + (cio - ps_c)).astype(jnp.int32)


def _prep3_body(key_ref, rank_ref, cnt_ref, soff_ref, pos_ref):
    i = pl.program_id(0)
    kio = lax.broadcasted_iota(jnp.int32, (JBLK, KPAD), 1)
    oh = (key_ref[...] == kio).astype(jnp.float32)
    rmask = (lax.broadcasted_iota(jnp.int32, (NJB, 1, 1), 0)
             < i).astype(jnp.float32)
    bp = jnp.sum(cnt_ref[...] * rmask, axis=0)
    base = soff_ref[...] + bp
    pos = jnp.sum(oh * base, axis=1, keepdims=True) + rank_ref[...]
    pos_ref[...] = pos.astype(jnp.int32)


@functools.cache
def _smesh():
    return plsc.VectorSubcoreMesh(core_axis_name="c", subcore_axis_name="s")


@functools.cache
def _make_sc_sortscat():
    jpt = C // _NS
    opw = C // _NW

    @functools.partial(
        pl.kernel, mesh=_smesh(),
        out_type=[
            jax.ShapeDtypeStruct((C,), jnp.int32),
            jax.ShapeDtypeStruct((C,), jnp.int32),
            jax.ShapeDtypeStruct((C,), jnp.float32),
        ],
        scratch_types=[
            pltpu.VMEM_SHARED((C,), jnp.int32),
            pltpu.VMEM_SHARED((C,), jnp.int32),
            pltpu.VMEM_SHARED((C,), jnp.float32),
            pltpu.VMEM((jpt,), jnp.int32),
            pltpu.VMEM((jpt,), jnp.int32),
            pltpu.VMEM((jpt,), jnp.int32),
            pltpu.VMEM((jpt,), jnp.float32),
            pltpu.VMEM((opw,), jnp.int32),
            pltpu.VMEM((opw,), jnp.float32),
            pltpu.SemaphoreType.DMA,
        ],
    )
    def _sc_sortscat(pos_h, src_h, dst_h, bas_h, src_o, dst_o, bas_o,
                     src_sp, dst_sp, bas_sp, pos_v, src_v, dst_v, bas_v,
                     bi_v, bf_v, sem):
        cid = lax.axis_index("c")
        sid = lax.axis_index("s")
        base = sid * jpt
        pltpu.sync_copy(pos_h.at[pl.ds(base, jpt)], pos_v)
        for q in range(jpt // 16):
            pv = pos_v[pl.ds(q * 16, 16)]
            pos_v[pl.ds(q * 16, 16)] = jnp.minimum(
                jnp.maximum(pv, 0), C - 1)
        pltpu.sync_copy(src_h.at[pl.ds(base, jpt)], src_v)
        pltpu.sync_copy(dst_h.at[pl.ds(base, jpt)], dst_v)
        pltpu.sync_copy(bas_h.at[pl.ds(base, jpt)], bas_v)
        pltpu.sync_copy(src_v, src_sp.at[pos_v])
        pltpu.sync_copy(dst_v, dst_sp.at[pos_v])
        pltpu.sync_copy(bas_v, bas_sp.at[pos_v])
        plsc.subcore_barrier()
        out0 = cid * (C // _NC) + sid * opw
        pltpu.sync_copy(src_sp.at[pl.ds(out0, opw)], bi_v)
        pltpu.sync_copy(bi_v, src_o.at[pl.ds(out0, opw)])
        pltpu.sync_copy(dst_sp.at[pl.ds(out0, opw)], bi_v)
        pltpu.sync_copy(bi_v, dst_o.at[pl.ds(out0, opw)])
        pltpu.sync_copy(bas_sp.at[pl.ds(out0, opw)], bf_v)
        pltpu.sync_copy(bf_v, bas_o.at[pl.ds(out0, opw)])

    return _sc_sortscat


@functools.cache
def _make_sc_gather():
    @functools.partial(
        pl.kernel, mesh=_smesh(),
        out_type=jax.ShapeDtypeStruct((C, KPAD), jnp.float32),
        scratch_types=[
            pltpu.VMEM((_JPW,), jnp.int32),
            pltpu.VMEM((_CH, KPAD), jnp.float32),
            pltpu.VMEM((_CH, KPAD), jnp.float32),
            pltpu.SemaphoreType.DMA,
            pltpu.SemaphoreType.DMA,
        ],
    )
    def _sc_gather(h_h, src_h, out_h, idx_v, buf0, buf1, sem, sem2):
        wid = lax.axis_index("s") * _NC + lax.axis_index("c")
        base = wid * _JPW
        bufs = [buf0, buf1]
        pltpu.sync_copy(src_h.at[pl.ds(base, _JPW)], idx_v)
        for q in range(_JPW // 16):
            idx_v[pl.ds(q * 16, 16)] = idx_v[pl.ds(q * 16, 16)] & (N - 1)
        gs = [None] * 8
        wbs = [None] * 8
        for ch in range(8):
            if ch >= 2:
                wbs[ch - 2].wait()
            gs[ch] = pltpu.async_copy(
                h_h.at[idx_v.at[pl.ds(ch * _CH, _CH)]], bufs[ch % 2], sem)
            if ch >= 1:
                gs[ch - 1].wait()
                wbs[ch - 1] = pltpu.async_copy(
                    bufs[(ch - 1) % 2],
                    out_h.at[pl.ds(base + (ch - 1) * _CH, _CH)], sem2)
        gs[7].wait()
        wbs[7] = pltpu.async_copy(
            bufs[7 % 2], out_h.at[pl.ds(base + 7 * _CH, _CH)], sem2)
        wbs[6].wait()
        wbs[7].wait()

    return _sc_gather


@functools.cache
def _make_sc_scatadd():
    return functools.partial(
        pl.kernel, mesh=_smesh(),
        out_type=jax.ShapeDtypeStruct((2, N, KPAD), jnp.float32),
        scratch_types=[
            pltpu.VMEM_SHARED((N, KPAD), jnp.float32),
            pltpu.VMEM((_CH, KPAD), jnp.float32),
            pltpu.VMEM((_CH, KPAD), jnp.float32),
            pltpu.VMEM((_CH,), jnp.int32),
            pltpu.VMEM((_CH,), jnp.int32),
            pltpu.SemaphoreType.DMA,
            pltpu.SemaphoreType.DMA,
        ],
    )(_sc_scatadd_body)


def _sc_scatadd_body(y_h, dst_h, zeros_h, out_h, acc_sh, ybuf0, ybuf1,
                     idx0, idx1, sem, sem2):
    cid = lax.axis_index("c")
    sid = lax.axis_index("s")
    wid = sid * _NC + cid
    pltpu.sync_copy(zeros_h.at[pl.ds(sid * _NPT, _NPT)],
                    acc_sh.at[pl.ds(sid * _NPT, _NPT)])
    plsc.subcore_barrier()
    base = wid * _JPW
    ybufs = [ybuf0, ybuf1]
    idxs = [idx0, idx1]
    adds = [None] * 8
    for ch in range(8):
        off = base + ch * _CH
        b = ybufs[ch % 2]
        ix = idxs[ch % 2]
        pltpu.sync_copy(dst_h.at[pl.ds(off, _CH)], ix)
        for q in range(_CH // 16):
            ix[pl.ds(q * 16, 16)] = ix[pl.ds(q * 16, 16)] & (N - 1)
        if ch >= 2:
            adds[ch - 2].wait()
        pltpu.async_copy(y_h.at[pl.ds(off, _CH)], b, sem).wait()
        adds[ch] = pltpu.async_copy(b, acc_sh.at[ix], sem2, add=True)
    adds[6].wait()
    adds[7].wait()
    plsc.subcore_barrier()
    pltpu.sync_copy(acc_sh.at[pl.ds(sid * _NPT, _NPT)],
                    out_h.at[cid, pl.ds(sid * _NPT, _NPT)])


def _mm_body(in_ch, tk_ref, vals_ref, bas_ref, W_ref, y_ref):
    t = pl.program_id(0)
    for j in range(8):
        k = tk_ref[t * 8 + j]
        A = (vals_ref[j * 128:(j + 1) * 128, :in_ch]
             * bas_ref[j * 128:(j + 1) * 128, :])
        w = W_ref[pl.ds(k * in_ch, in_ch), :]
        yv = jnp.dot(A, w, preferred_element_type=jnp.float32)
        y_ref[j * 128:(j + 1) * 128, :] = jnp.concatenate(
            [yv, jnp.zeros((128, KPAD - F), jnp.float32)], axis=1)


def _combine_body(p_ref, b_ref, o_ref):
    o_ref[...] = jnp.maximum(p_ref[0] + p_ref[1] + b_ref[...], 0.0)


def _final_body(x_ref, h1, h2, h3, h4, h5, h6,
                lw0, lw1, lw2, lw3, lw4, lw5, lw6, lb_ref, o_ref):
    hs = [x_ref, h1, h2, h3, h4, h5, h6]
    lws = [lw0, lw1, lw2, lw3, lw4, lw5, lw6]
    acc = jnp.zeros((F, N), jnp.float32)
    for h_ref, lw_ref in zip(hs, lws):
        acc += lax.dot_general(lw_ref[...], h_ref[...],
                               (((0,), (1,)), ((), ())),
                               preferred_element_type=jnp.float32)
    o_ref[...] = acc + lb_ref[...]


def kernel(x, edge_index, edge_attr, w0, root0, b0, w1, root1, b1,
           w2, root2, b2, w3, root3, b3, w4, root4, b4, w5, root5, b5,
           lin_w, lin_b):
    f32 = jnp.float32
    i32 = jnp.int32
    x_pad = jnp.pad(x.astype(f32), ((0, 0), (0, KPAD - x.shape[1])))
    attr_t = edge_attr.astype(f32).T
    ei = edge_index.astype(i32)

    def flat_w(w, r, in_p):
        kp = jnp.zeros((KPAD, in_p, F), f32)
        kp = kp.at[:K, :w.shape[1], :].set(w.astype(f32))
        kp = kp.at[K, :r.shape[0], :].set(r.astype(f32))
        return kp.reshape(KPAD * in_p, F)

    Ws = [flat_w(w0, root0, IN0P)] + [
        flat_w(w, r, F) for w, r in
        ((w1, root1), (w2, root2), (w3, root3), (w4, root4), (w5, root5))]
    Bs = [jnp.pad(b.astype(f32).reshape(1, F), ((0, 0), (0, KPAD - F)))
          for b in (b0, b1, b2, b3, b4, b5)]

    lw0p = jnp.zeros((KPAD, F), f32).at[:x.shape[1], :].set(
        lin_w[:x.shape[1], :].astype(f32))
    lws = [lw0p] + [
        jnp.zeros((KPAD, F), f32).at[:F, :].set(
            lin_w[x.shape[1] + l * F:x.shape[1] + (l + 1) * F,
                  :].astype(f32)) for l in range(LAYERS)]
    lb_col = lin_b.astype(f32).reshape(F, 1)

    ts2048 = jnp.tril(jnp.ones((JBLK, JBLK), f32), -1)
    ts128 = jnp.triu(jnp.ones((KPAD, KPAD), f32), 1)
    zeros_nf = jnp.zeros((N, KPAD), f32)

    neb = E // EBLK
    basT, widxT, srep, drep, deg = pl.pallas_call(
        _prep1_body,
        grid=(neb,),
        in_specs=[pl.BlockSpec((2, EBLK), lambda i: (0, i)),
                  pl.BlockSpec((3, EBLK), lambda i: (0, i))],
        out_specs=[pl.BlockSpec((EBLK, 8), lambda i: (i, 0)),
                   pl.BlockSpec((EBLK, 8), lambda i: (i, 0)),
                   pl.BlockSpec((EBLK, 8), lambda i: (i, 0)),
                   pl.BlockSpec((EBLK, 8), lambda i: (i, 0)),
                   pl.BlockSpec((N, 1), lambda i: (0, 0))],
        out_shape=[jax.ShapeDtypeStruct((E, 8), f32),
                   jax.ShapeDtypeStruct((E, 8), i32),
                   jax.ShapeDtypeStruct((E, 8), i32),
                   jax.ShapeDtypeStruct((E, 8), i32),
                   jax.ShapeDtypeStruct((N, 1), f32)],
        compiler_params=_VM,
    )(ei, attr_t)

    basf = pl.pallas_call(
        _prep1b_body,
        grid=(neb,),
        in_specs=[pl.BlockSpec((2, EBLK), lambda i: (0, i)),
                  pl.BlockSpec((N, 1), lambda i: (0, 0)),
                  pl.BlockSpec((EBLK, 8), lambda i: (i, 0))],
        out_specs=pl.BlockSpec((EBLK, 8), lambda i: (i, 0)),
        out_shape=jax.ShapeDtypeStruct((E, 8), f32),
        compiler_params=_VM,
    )(ei, deg, basT)

    key_col = widxT.reshape(J, 1)
    rank, cnt3 = pl.pallas_call(
        _prep2_body,
        grid=(NJB,),
        in_specs=[pl.BlockSpec((JBLK, 1), lambda i: (i, 0)),
                  pl.BlockSpec((JBLK, JBLK), lambda i: (0, 0))],
        out_specs=[pl.BlockSpec((JBLK, 1), lambda i: (i, 0)),
                   pl.BlockSpec((1, 1, KPAD), lambda i: (i, 0, 0))],
        out_shape=[jax.ShapeDtypeStruct((J, 1), f32),
                   jax.ShapeDtypeStruct((NJB, 1, KPAD), f32)],
        compiler_params=_VM,
    )(key_col, ts2048)

    soff, tkmap, pos_pad = pl.pallas_call(
        _prep4_body,
        in_specs=[pl.BlockSpec((NJB, 1, KPAD), lambda: (0, 0, 0)),
                  pl.BlockSpec((KPAD, KPAD), lambda: (0, 0))],
        out_specs=[pl.BlockSpec((1, KPAD), lambda: (0, 0)),
                   pl.BlockSpec((T_TOT + 112, 1), lambda: (0, 0)),
                   pl.BlockSpec((NPAD, 1), lambda: (0, 0))],
        out_shape=[jax.ShapeDtypeStruct((1, KPAD), f32),
                   jax.ShapeDtypeStruct((T_TOT + 112, 1), i32),
                   jax.ShapeDtypeStruct((NPAD, 1), i32)],
        compiler_params=_VM,
    )(cnt3, ts128)

    pos_main = pl.pallas_call(
        _prep3_body,
        grid=(NJB,),
        in_specs=[pl.BlockSpec((JBLK, 1), lambda i: (i, 0)),
                  pl.BlockSpec((JBLK, 1), lambda i: (i, 0)),
                  pl.BlockSpec((NJB, 1, KPAD), lambda i: (0, 0, 0)),
                  pl.BlockSpec((1, KPAD), lambda i: (0, 0))],
        out_specs=pl.BlockSpec((JBLK, 1), lambda i: (i, 0)),
        out_shape=jax.ShapeDtypeStruct((J, 1), i32),
        compiler_params=_VM,
    )(key_col, rank, cnt3, soff)

    sd_pad = (jnp.arange(NPAD, dtype=i32) & (N - 1))
    ar_n = jnp.arange(N, dtype=i32)
    pos_ext = jnp.concatenate(
        [pos_main.reshape(J), pos_pad.reshape(NPAD), ar_n])
    src_ext = jnp.concatenate([srep.reshape(J), sd_pad, ar_n])
    dst_ext = jnp.concatenate([drep.reshape(J), sd_pad, ar_n])
    bas_ext = jnp.concatenate(
        [basf.reshape(J), jnp.zeros((NPAD,), f32), jnp.ones((N,), f32)])

    src_s, dst_s, bas_s = _make_sc_sortscat()(pos_ext, src_ext, dst_ext, bas_ext)
    bas_col = bas_s.reshape(C, 1)
    tk = tkmap.reshape(T_TOT + 112)[:T_TOT]

    h = x_pad
    feats = []
    for l in range(LAYERS):
        in_ch = IN0P if l == 0 else F
        vals = _make_sc_gather()(h, src_s)
        y = pl.pallas_call(
            functools.partial(_mm_body, in_ch),
            grid_spec=pltpu.PrefetchScalarGridSpec(
                num_scalar_prefetch=1,
                grid=(T_TOT // 8,),
                in_specs=[
                    pl.BlockSpec((1024, KPAD), lambda t, tk_: (t, 0)),
                    pl.BlockSpec((1024, 1), lambda t, tk_: (t, 0)),
                    pl.BlockSpec((KPAD * in_ch, F), lambda t, tk_: (0, 0)),
                ],
                out_specs=pl.BlockSpec((1024, KPAD), lambda t, tk_: (t, 0)),
            ),
            out_shape=jax.ShapeDtypeStruct((C, KPAD), f32),
            compiler_params=_VM,
        )(tk, vals, bas_col, Ws[l])
        parts = _make_sc_scatadd()(y, dst_s, zeros_nf)
        h = pl.pallas_call(
            _combine_body,
            in_specs=[pl.BlockSpec((2, N, KPAD), lambda: (0, 0, 0)),
                      pl.BlockSpec((1, KPAD), lambda: (0, 0))],
            out_specs=pl.BlockSpec((N, KPAD), lambda: (0, 0)),
            out_shape=jax.ShapeDtypeStruct((N, KPAD), f32),
            compiler_params=_VM,
        )(parts, Bs[l])
        feats.append(h)

    out = pl.pallas_call(
        _final_body,
        in_specs=[pl.BlockSpec((N, KPAD), lambda: (0, 0))] * (LAYERS + 1)
        + [pl.BlockSpec((KPAD, F), lambda: (0, 0))] * (LAYERS + 1)
        + [pl.BlockSpec((F, 1), lambda: (0, 0))],
        out_specs=pl.BlockSpec((F, N), lambda: (0, 0)),
        out_shape=jax.ShapeDtypeStruct((F, N), f32),
        compiler_params=_VM,
    )(x_pad, *feats, *lws, lb_col)
    return out
